# baseline (speedup 1.0000x reference)
import jax
import jax.numpy as jnp
from jax.experimental import pallas as pl
from jax.experimental.pallas import tpu as pltpu

HIDDEN = 128
N_AA = 20
PEP_LENGTH = 15
MAX_TCR_LEN = 27
VOCAB = N_AA + 1
ENC_DIM = 32 + N_AA
VOC_PAD = 32
ENC_PAD = 64
G4 = 4 * HIDDEN
NB = 2


def _sig(x):
    return 0.5 + 0.5 * jnp.tanh(0.5 * x)


def _cell(gates, c):
    i = _sig(gates[:HIDDEN])
    f = _sig(gates[HIDDEN:2 * HIDDEN])
    g = jnp.tanh(gates[2 * HIDDEN:3 * HIDDEN])
    o = _sig(gates[3 * HIDDEN:])
    c_new = f * c + i * g
    h_new = o * jnp.tanh(c_new)
    return h_new, c_new


def _dot(a, b):
    return jnp.dot(a.astype(jnp.bfloat16), b.astype(jnp.bfloat16),
                   preferred_element_type=jnp.float32)


def _seq_kernel(pep_t3_ref, tcr_t3_ref, encT_ref,
                wih_pf_ref, wih_pb_ref, wih_tf_ref, wih_tb_ref,
                whh_pf_ref, whh_pb_ref, whh_tf_ref, whh_tb_ref,
                h0p_ref, c0p_ref, h0t_ref, c0t_ref,
                tcr_out_ref, tcr_hn_ref, pep_emb_ref,
                oh_pep_ref, oh_tcr_ref):
    encT = encT_ref[...]
    tbl_pf = _dot(wih_pf_ref[...], encT)
    tbl_pb = _dot(wih_pb_ref[...], encT)
    tbl_tf = _dot(wih_tf_ref[...], encT)
    tbl_tb = _dot(wih_tb_ref[...], encT)

    Bb = pep_t3_ref.shape[2]
    iota_p = jax.lax.broadcasted_iota(jnp.int32, (PEP_LENGTH, VOC_PAD, Bb), 1)
    oh_pep_ref[...] = (pep_t3_ref[...] == iota_p).astype(jnp.float32)
    iota_t = jax.lax.broadcasted_iota(jnp.int32, (MAX_TCR_LEN, VOC_PAD, Bb), 1)
    oh_tcr_ref[...] = (tcr_t3_ref[...] == iota_t).astype(jnp.float32)

    lens_p = jnp.sum((pep_t3_ref[:, 0, :] != 0).astype(jnp.int32), axis=0,
                     keepdims=True)
    lens_t = jnp.sum((tcr_t3_ref[:, 0, :] != 0).astype(jnp.int32), axis=0,
                     keepdims=True)

    def cell_step(oh, tbl, w, h, c, m):
        g = _dot(tbl, oh) + _dot(w, h)
        h_new, c_new = _cell(g, c)
        return jnp.where(m, h_new, h), jnp.where(m, c_new, c), h_new

    wpf, wpb = whh_pf_ref[...], whh_pb_ref[...]
    wtf, wtb = whh_tf_ref[...], whh_tb_ref[...]

    def tcr_step(i, hft, cft, hbt, cbt):
        tb = MAX_TCR_LEN - 1 - i
        mf = i < lens_t
        mb = tb < lens_t
        hft, cft, hf_new = cell_step(oh_tcr_ref[i], tbl_tf, wtf, hft, cft, mf)
        hbt, cbt, hb_new = cell_step(oh_tcr_ref[tb], tbl_tb, wtb, hbt, cbt, mb)
        tcr_out_ref[i, :HIDDEN, :] = jnp.where(mf, hf_new, 0.0)
        tcr_out_ref[tb, HIDDEN:, :] = jnp.where(mb, hb_new, 0.0)
        return hft, cft, hbt, cbt

    def body_a(i, carry):
        hfp, cfp, hbp, cbp, hft, cft, hbt, cbt = carry
        tb = PEP_LENGTH - 1 - i
        hfp, cfp, _ = cell_step(oh_pep_ref[i], tbl_pf, wpf, hfp, cfp,
                                i < lens_p)
        hbp, cbp, _ = cell_step(oh_pep_ref[tb], tbl_pb, wpb, hbp, cbp,
                                tb < lens_p)
        hft, cft, hbt, cbt = tcr_step(i, hft, cft, hbt, cbt)
        return hfp, cfp, hbp, cbp, hft, cft, hbt, cbt

    def body_b(i, carry):
        hft, cft, hbt, cbt = carry
        return tcr_step(i, hft, cft, hbt, cbt)

    h0p, c0p = h0p_ref[...], c0p_ref[...]
    h0t, c0t = h0t_ref[...], c0t_ref[...]
    hfp, _, hbp, _, hft, cft, hbt, cbt = jax.lax.fori_loop(
        0, PEP_LENGTH, body_a,
        (h0p, c0p, h0p, c0p, h0t, c0t, h0t, c0t), unroll=3)
    hft, _, hbt, _ = jax.lax.fori_loop(
        PEP_LENGTH, MAX_TCR_LEN, body_b, (hft, cft, hbt, cbt), unroll=3)

    pep_emb_ref[:HIDDEN, :] = hfp
    pep_emb_ref[HIDDEN:, :] = hbp
    tcr_hn_ref[0] = hft
    tcr_hn_ref[1] = hbt


def _prep_w(wih, b):
    w = jnp.zeros((G4, ENC_PAD), jnp.float32)
    return w.at[:, :ENC_DIM].set(wih).at[:, ENC_DIM].set(b)


@jax.jit
def kernel(obs, emb_table, onehot_dict, pep_Wih_f, pep_Whh_f, pep_b_f,
           pep_Wih_b, pep_Whh_b, pep_b_b, tcr_Wih_f, tcr_Whh_f, tcr_b_f,
           tcr_Wih_b, tcr_Whh_b, tcr_b_b, h0_pep, c0_pep, h0_tcr, c0_tcr):
    B = obs.shape[0]
    Bb = B // NB
    obs = obs.astype(jnp.int32)
    tcr_t3 = obs[:, :MAX_TCR_LEN].T.reshape(MAX_TCR_LEN, 1, B)
    pep_t3 = obs[:, MAX_TCR_LEN:].T.reshape(PEP_LENGTH, 1, B)

    encT = jnp.zeros((ENC_PAD, VOC_PAD), jnp.float32)
    encT = encT.at[:ENC_DIM, :VOCAB].set(
        jnp.concatenate([emb_table, onehot_dict], axis=1).T)
    encT = encT.at[ENC_DIM, :].set(1.0)

    args = (pep_t3, tcr_t3, encT,
            _prep_w(pep_Wih_f, pep_b_f), _prep_w(pep_Wih_b, pep_b_b),
            _prep_w(tcr_Wih_f, tcr_b_f), _prep_w(tcr_Wih_b, tcr_b_b),
            pep_Whh_f, pep_Whh_b, tcr_Whh_f, tcr_Whh_b,
            h0_pep.T, c0_pep.T, h0_tcr.T, c0_tcr.T)

    full = lambda b: (0, 0)
    bat2 = lambda b: (0, b)
    bat3 = lambda b: (0, 0, b)
    in_specs = [
        pl.BlockSpec((PEP_LENGTH, 1, Bb), bat3),
        pl.BlockSpec((MAX_TCR_LEN, 1, Bb), bat3),
        pl.BlockSpec((ENC_PAD, VOC_PAD), full),
        pl.BlockSpec((G4, ENC_PAD), full),
        pl.BlockSpec((G4, ENC_PAD), full),
        pl.BlockSpec((G4, ENC_PAD), full),
        pl.BlockSpec((G4, ENC_PAD), full),
        pl.BlockSpec((G4, HIDDEN), full),
        pl.BlockSpec((G4, HIDDEN), full),
        pl.BlockSpec((G4, HIDDEN), full),
        pl.BlockSpec((G4, HIDDEN), full),
        pl.BlockSpec((HIDDEN, Bb), bat2),
        pl.BlockSpec((HIDDEN, Bb), bat2),
        pl.BlockSpec((HIDDEN, Bb), bat2),
        pl.BlockSpec((HIDDEN, Bb), bat2),
    ]
    out_specs = [
        pl.BlockSpec((MAX_TCR_LEN, 2 * HIDDEN, Bb), bat3),
        pl.BlockSpec((2, HIDDEN, Bb), bat3),
        pl.BlockSpec((2 * HIDDEN, Bb), bat2),
    ]
    out_shapes = [
        jax.ShapeDtypeStruct((MAX_TCR_LEN, 2 * HIDDEN, B), jnp.float32),
        jax.ShapeDtypeStruct((2, HIDDEN, B), jnp.float32),
        jax.ShapeDtypeStruct((2 * HIDDEN, B), jnp.float32),
    ]
    tcr_out_k, tcr_hn_k, pep_emb_k = pl.pallas_call(
        _seq_kernel,
        grid=(NB,),
        in_specs=in_specs,
        out_specs=out_specs,
        out_shape=out_shapes,
        scratch_shapes=[
            pltpu.VMEM((PEP_LENGTH, VOC_PAD, Bb), jnp.float32),
            pltpu.VMEM((MAX_TCR_LEN, VOC_PAD, Bb), jnp.float32),
        ],
        compiler_params=pltpu.CompilerParams(
            dimension_semantics=("parallel",)),
    )(*args)
    tcr_out = jnp.transpose(tcr_out_k, (2, 0, 1))
    tcr_hn = jnp.transpose(tcr_hn_k, (0, 2, 1))
    pep_emb = pep_emb_k.T
    return tcr_out, tcr_hn, pep_emb


# speedup vs baseline: 1.0092x; 1.0092x over previous
import jax
import jax.numpy as jnp
from jax.experimental import pallas as pl
from jax.experimental.pallas import tpu as pltpu

HIDDEN = 128
N_AA = 20
PEP_LENGTH = 15
MAX_TCR_LEN = 27
VOCAB = N_AA + 1
ENC_DIM = 32 + N_AA
VOC_PAD = 32
ENC_PAD = 64
G4 = 4 * HIDDEN
NB = 2


def _sig(x):
    return 0.5 + 0.5 * jnp.tanh(0.5 * x)


def _cell(gates, c):
    i = _sig(gates[:HIDDEN])
    f = _sig(gates[HIDDEN:2 * HIDDEN])
    g = jnp.tanh(gates[2 * HIDDEN:3 * HIDDEN])
    o = _sig(gates[3 * HIDDEN:])
    c_new = f * c + i * g
    h_new = o * jnp.tanh(c_new)
    return h_new, c_new


def _dot(a, b):
    return jnp.dot(a, b, preferred_element_type=jnp.float32)


def _seq_kernel(pep_t3_ref, tcr_t3_ref, encT_ref,
                wih_pf_ref, wih_pb_ref, wih_tf_ref, wih_tb_ref,
                whh_pf_ref, whh_pb_ref, whh_tf_ref, whh_tb_ref,
                h0p_ref, c0p_ref, h0t_ref, c0t_ref,
                tcr_out_ref, tcr_hn_ref, pep_emb_ref,
                oh_pep_ref, oh_tcr_ref):
    encT = encT_ref[...]
    tbl_pf = _dot(wih_pf_ref[...], encT)
    tbl_pb = _dot(wih_pb_ref[...], encT)
    tbl_tf = _dot(wih_tf_ref[...], encT)
    tbl_tb = _dot(wih_tb_ref[...], encT)

    Bb = pep_t3_ref.shape[2]
    iota_p = jax.lax.broadcasted_iota(jnp.int32, (PEP_LENGTH, VOC_PAD, Bb), 1)
    oh_pep_ref[...] = (pep_t3_ref[...] == iota_p).astype(jnp.float32)
    iota_t = jax.lax.broadcasted_iota(jnp.int32, (MAX_TCR_LEN, VOC_PAD, Bb), 1)
    oh_tcr_ref[...] = (tcr_t3_ref[...] == iota_t).astype(jnp.float32)

    lens_p = jnp.sum((pep_t3_ref[:, 0, :] != 0).astype(jnp.int32), axis=0,
                     keepdims=True)
    lens_t = jnp.sum((tcr_t3_ref[:, 0, :] != 0).astype(jnp.int32), axis=0,
                     keepdims=True)

    def cell_step(oh, tbl, w, h, c, m):
        g = _dot(tbl, oh) + _dot(w, h)
        h_new, c_new = _cell(g, c)
        return jnp.where(m, h_new, h), jnp.where(m, c_new, c), h_new

    wpf, wpb = whh_pf_ref[...], whh_pb_ref[...]
    wtf, wtb = whh_tf_ref[...], whh_tb_ref[...]

    def tcr_step(i, hft, cft, hbt, cbt):
        tb = MAX_TCR_LEN - 1 - i
        mf = i < lens_t
        mb = tb < lens_t
        hft, cft, hf_new = cell_step(oh_tcr_ref[i], tbl_tf, wtf, hft, cft, mf)
        hbt, cbt, hb_new = cell_step(oh_tcr_ref[tb], tbl_tb, wtb, hbt, cbt, mb)
        tcr_out_ref[i, :HIDDEN, :] = jnp.where(mf, hf_new, 0.0)
        tcr_out_ref[tb, HIDDEN:, :] = jnp.where(mb, hb_new, 0.0)
        return hft, cft, hbt, cbt

    def body_a(i, carry):
        hfp, cfp, hbp, cbp, hft, cft, hbt, cbt = carry
        tb = PEP_LENGTH - 1 - i
        hfp, cfp, _ = cell_step(oh_pep_ref[i], tbl_pf, wpf, hfp, cfp,
                                i < lens_p)
        hbp, cbp, _ = cell_step(oh_pep_ref[tb], tbl_pb, wpb, hbp, cbp,
                                tb < lens_p)
        hft, cft, hbt, cbt = tcr_step(i, hft, cft, hbt, cbt)
        return hfp, cfp, hbp, cbp, hft, cft, hbt, cbt

    def body_b(i, carry):
        hft, cft, hbt, cbt = carry
        return tcr_step(i, hft, cft, hbt, cbt)

    h0p, c0p = h0p_ref[...], c0p_ref[...]
    h0t, c0t = h0t_ref[...], c0t_ref[...]
    hfp, _, hbp, _, hft, cft, hbt, cbt = jax.lax.fori_loop(
        0, PEP_LENGTH, body_a,
        (h0p, c0p, h0p, c0p, h0t, c0t, h0t, c0t), unroll=3)
    hft, _, hbt, _ = jax.lax.fori_loop(
        PEP_LENGTH, MAX_TCR_LEN, body_b, (hft, cft, hbt, cbt), unroll=3)

    pep_emb_ref[:HIDDEN, :] = hfp
    pep_emb_ref[HIDDEN:, :] = hbp
    tcr_hn_ref[0] = hft
    tcr_hn_ref[1] = hbt


def _prep_w(wih, b):
    w = jnp.zeros((G4, ENC_PAD), jnp.float32)
    return w.at[:, :ENC_DIM].set(wih).at[:, ENC_DIM].set(b)


@jax.jit
def kernel(obs, emb_table, onehot_dict, pep_Wih_f, pep_Whh_f, pep_b_f,
           pep_Wih_b, pep_Whh_b, pep_b_b, tcr_Wih_f, tcr_Whh_f, tcr_b_f,
           tcr_Wih_b, tcr_Whh_b, tcr_b_b, h0_pep, c0_pep, h0_tcr, c0_tcr):
    B = obs.shape[0]
    Bb = B // NB
    obs = obs.astype(jnp.int32)
    tcr_t3 = obs[:, :MAX_TCR_LEN].T.reshape(MAX_TCR_LEN, 1, B)
    pep_t3 = obs[:, MAX_TCR_LEN:].T.reshape(PEP_LENGTH, 1, B)

    encT = jnp.zeros((ENC_PAD, VOC_PAD), jnp.float32)
    encT = encT.at[:ENC_DIM, :VOCAB].set(
        jnp.concatenate([emb_table, onehot_dict], axis=1).T)
    encT = encT.at[ENC_DIM, :].set(1.0)

    args = (pep_t3, tcr_t3, encT,
            _prep_w(pep_Wih_f, pep_b_f), _prep_w(pep_Wih_b, pep_b_b),
            _prep_w(tcr_Wih_f, tcr_b_f), _prep_w(tcr_Wih_b, tcr_b_b),
            pep_Whh_f, pep_Whh_b, tcr_Whh_f, tcr_Whh_b,
            h0_pep.T, c0_pep.T, h0_tcr.T, c0_tcr.T)

    full = lambda b: (0, 0)
    bat2 = lambda b: (0, b)
    bat3 = lambda b: (0, 0, b)
    in_specs = [
        pl.BlockSpec((PEP_LENGTH, 1, Bb), bat3),
        pl.BlockSpec((MAX_TCR_LEN, 1, Bb), bat3),
        pl.BlockSpec((ENC_PAD, VOC_PAD), full),
        pl.BlockSpec((G4, ENC_PAD), full),
        pl.BlockSpec((G4, ENC_PAD), full),
        pl.BlockSpec((G4, ENC_PAD), full),
        pl.BlockSpec((G4, ENC_PAD), full),
        pl.BlockSpec((G4, HIDDEN), full),
        pl.BlockSpec((G4, HIDDEN), full),
        pl.BlockSpec((G4, HIDDEN), full),
        pl.BlockSpec((G4, HIDDEN), full),
        pl.BlockSpec((HIDDEN, Bb), bat2),
        pl.BlockSpec((HIDDEN, Bb), bat2),
        pl.BlockSpec((HIDDEN, Bb), bat2),
        pl.BlockSpec((HIDDEN, Bb), bat2),
    ]
    out_specs = [
        pl.BlockSpec((MAX_TCR_LEN, 2 * HIDDEN, Bb), bat3),
        pl.BlockSpec((2, HIDDEN, Bb), bat3),
        pl.BlockSpec((2 * HIDDEN, Bb), bat2),
    ]
    out_shapes = [
        jax.ShapeDtypeStruct((MAX_TCR_LEN, 2 * HIDDEN, B), jnp.float32),
        jax.ShapeDtypeStruct((2, HIDDEN, B), jnp.float32),
        jax.ShapeDtypeStruct((2 * HIDDEN, B), jnp.float32),
    ]
    tcr_out_k, tcr_hn_k, pep_emb_k = pl.pallas_call(
        _seq_kernel,
        grid=(NB,),
        in_specs=in_specs,
        out_specs=out_specs,
        out_shape=out_shapes,
        scratch_shapes=[
            pltpu.VMEM((PEP_LENGTH, VOC_PAD, Bb), jnp.float32),
            pltpu.VMEM((MAX_TCR_LEN, VOC_PAD, Bb), jnp.float32),
        ],
        compiler_params=pltpu.CompilerParams(
            dimension_semantics=("parallel",)),
    )(*args)
    tcr_out = jnp.transpose(tcr_out_k, (2, 0, 1))
    tcr_hn = jnp.transpose(tcr_hn_k, (0, 2, 1))
    pep_emb = pep_emb_k.T
    return tcr_out, tcr_hn, pep_emb
